# Initial kernel scaffold; baseline (speedup 1.0000x reference)
import jax
import jax.numpy as jnp
from jax.experimental import pallas as pl


def kernel(x0, x1, W_mlp, b_mlp, W1, b1, W2, b2, W3, b3, Wp, bp, Wg, bg, edge_index, edge_index_g):
    raise NotImplementedError("write your pallas kernel here")


# speedup vs baseline: 12.9696x; 12.9696x over previous
import functools

import jax
import jax.numpy as jnp
from jax import lax
from jax.experimental import pallas as pl
from jax.experimental.pallas import tpu as pltpu
from jax.experimental.pallas import tpu_sc as plsc

N = 10000
E = 320000
F = 128
EG = 2048

NC = 2
NS = 16
NW = NC * NS

EPT = E // NW
K = 80
NCH = EPT // K

EGPT = EG // NW
GBINS = F * F
NPAD = 10240

@functools.cache
def _mesh():
    return plsc.VectorSubcoreMesh(core_axis_name="c", subcore_axis_name="s",
                                  num_cores=NC, num_subcores=NS)


def _zero_fill(ref, nrows, ncols):
    zer = jnp.zeros((16,), jnp.float32)

    def body(i, _):
        for j in range(ncols // 16):
            ref[i, pl.ds(j * 16, 16)] = zer
        return 0

    lax.fori_loop(0, nrows, body, 0)


def _zero_fill_1d(ref, n):
    zer = jnp.zeros((16,), jnp.float32)

    def body(i, _):
        ref[pl.ds(i * 16, 16)] = zer
        return 0

    lax.fori_loop(0, n // 16, body, 0)


def _ones_fill_1d(ref, n):
    one = jnp.full((16,), 1.0, jnp.float32)
    for j in range(n // 16):
        ref[pl.ds(j * 16, 16)] = one


@functools.cache
def _get_sc_hist():
  return pl.kernel(
    _sc_hist_body,
    out_type=[
        jax.ShapeDtypeStruct((NC, NPAD), jnp.float32),
        jax.ShapeDtypeStruct((NC, GBINS), jnp.float32),
    ],
    mesh=_mesh(),
    scratch_types=[
        pltpu.VMEM_SHARED((NPAD,), jnp.float32),
        pltpu.VMEM_SHARED((GBINS,), jnp.float32),
        pltpu.VMEM((1024,), jnp.float32),
        pltpu.VMEM((K,), jnp.int32),
        pltpu.VMEM((K,), jnp.float32),
        pltpu.VMEM((EGPT,), jnp.int32),
        pltpu.VMEM((EGPT,), jnp.int32),
        pltpu.VMEM((EGPT,), jnp.int32),
        pltpu.VMEM((EGPT,), jnp.float32),
        pltpu.VMEM((1024,), jnp.float32),
    ],
  )


def _sc_hist_body(dst_hbm, gsrc_hbm, gdst_hbm, deg_out, gcnt_out,
             sh_deg, sh_g, zbuf, idx_v, ones_v, gs_v, gd_v, gf_v, gones_v,
             obuf):
    cid = lax.axis_index("c")
    sid = lax.axis_index("s")
    wid = cid * NS + sid

    _zero_fill_1d(zbuf, 1024)
    _ones_fill_1d(ones_v, K)
    _ones_fill_1d(gones_v, EGPT)

    @pl.when(sid < 10)
    def _():
        pltpu.sync_copy(zbuf, sh_deg.at[pl.ds(sid * 1024, 1024)])

    pltpu.sync_copy(zbuf, sh_g.at[pl.ds(sid * 1024, 1024)])
    plsc.subcore_barrier()

    base = wid * EPT

    def chunk(c, _):
        off = pl.multiple_of(base + c * K, 8)
        pltpu.sync_copy(dst_hbm.at[pl.ds(off, K)], idx_v)
        pltpu.sync_copy(ones_v, sh_deg.at[idx_v], add=True)
        return 0

    lax.fori_loop(0, NCH, chunk, 0)

    goff = pl.multiple_of(wid * EGPT, 8)
    pltpu.sync_copy(gsrc_hbm.at[pl.ds(goff, EGPT)], gs_v)
    pltpu.sync_copy(gdst_hbm.at[pl.ds(goff, EGPT)], gd_v)
    for j in range(EGPT // 16):
        s = gs_v[pl.ds(j * 16, 16)]
        d = gd_v[pl.ds(j * 16, 16)]
        gf_v[pl.ds(j * 16, 16)] = d * F + s
    pltpu.sync_copy(gones_v, sh_g.at[gf_v], add=True)

    plsc.subcore_barrier()

    @pl.when(sid < 10)
    def _():
        pltpu.sync_copy(sh_deg.at[pl.ds(sid * 1024, 1024)], obuf)
        pltpu.sync_copy(obuf, deg_out.at[cid, pl.ds(sid * 1024, 1024)])

    pltpu.sync_copy(sh_g.at[pl.ds(sid * 1024, 1024)], obuf)
    pltpu.sync_copy(obuf, gcnt_out.at[cid, pl.ds(sid * 1024, 1024)])


ZR = 200

@functools.cache
def _get_sc_agg():
  return pl.kernel(
    _sc_agg_body,
    out_type=jax.ShapeDtypeStruct((NC, N, F), jnp.float32),
    mesh=_mesh(),
    scratch_types=[
        pltpu.VMEM_SHARED((N, F), jnp.float32),
        pltpu.VMEM((ZR, F), jnp.float32),
        pltpu.VMEM((K,), jnp.int32),
        pltpu.VMEM((K,), jnp.int32),
        pltpu.VMEM((K, F), jnp.float32),
        pltpu.SemaphoreType.DMA,
    ],
  )


def _sc_agg_body(h_hbm, src_hbm, dst_hbm, agg_out, acc, stage, src_v, dst_v,
            rows_v, sem):
    cid = lax.axis_index("c")
    sid = lax.axis_index("s")
    wid = cid * NS + sid

    _zero_fill(stage, ZR, F)
    row0 = sid * 1000

    @pl.when(sid < 10)
    def _():
        for t in range(5):
            pltpu.sync_copy(stage, acc.at[pl.ds(row0 + t * ZR, ZR)])

    plsc.subcore_barrier()

    base = wid * EPT

    def chunk(c, _):
        off = pl.multiple_of(base + c * K, 8)
        pltpu.sync_copy(src_hbm.at[pl.ds(off, K)], src_v)
        pltpu.sync_copy(dst_hbm.at[pl.ds(off, K)], dst_v)
        pltpu.async_copy(h_hbm.at[src_v], rows_v, sem).wait()
        pltpu.sync_copy(rows_v, acc.at[dst_v], add=True)
        return 0

    lax.fori_loop(0, NCH, chunk, 0)
    plsc.subcore_barrier()

    @pl.when(sid < 10)
    def _():
        for t in range(5):
            r = row0 + t * ZR
            pltpu.sync_copy(acc.at[pl.ds(r, ZR)], stage)
            pltpu.sync_copy(stage, agg_out.at[cid, pl.ds(r, ZR)])


def _sc_hist(dst, gsrc, gdst):
    return _get_sc_hist()(dst, gsrc, gdst)


def _sc_agg(h, src, dst):
    return _get_sc_agg()(h, src, dst)


BR = 1000
GRID = N // BR

def _dot(a, b):
    return jnp.dot(a, b, preferred_element_type=jnp.float32)


def _l2n(x):
    n = jnp.sqrt(jnp.sum(x * x, axis=1, keepdims=True))
    return x / jnp.maximum(n, 1e-12)


def _lrelu(v):
    return jnp.where(v >= 0, v, 0.01 * v)


def _tc_b_kernel(x0_ref, x1_ref, wm_ref, bm_ref, w1_ref, wp_ref, bp_ref,
                 wg_ref, dp_ref, emb1_ref, p1_ref, hs1_ref, dinv_ref,
                 hg_ref):
    i = pl.program_id(0)
    deg = dp_ref[0] + dp_ref[1] + 1.0
    dinv = lax.rsqrt(deg)
    dinv_ref[...] = dinv
    h1 = jax.nn.relu(_dot(x0_ref[...], wm_ref[...]) + bm_ref[...])
    emb1_ref[...] = _l2n(h1)
    p1_ref[...] = _dot(h1, wp_ref[...]) + bp_ref[...]
    hs1_ref[...] = dinv * _dot(x1_ref[...], w1_ref[...])

    @pl.when(i == 0)
    def _():
        hg_ref[...] = jnp.zeros((F, F), jnp.float32)

    hg_ref[...] += lax.dot_general(
        x0_ref[...], wg_ref[...], (((0,), (0,)), ((), ())),
        preferred_element_type=jnp.float32)


def _tc_b(x0, x1, wm, bm, w1, wp, bp, wg, dparts):
    row = lambda i: (i, 0)
    full = lambda i: (0, 0)
    return pl.pallas_call(
        _tc_b_kernel,
        grid=(GRID,),
        in_specs=[
            pl.BlockSpec((BR, F), row),
            pl.BlockSpec((BR, F), row),
            pl.BlockSpec((F, F), full),
            pl.BlockSpec((1, F), full),
            pl.BlockSpec((F, F), full),
            pl.BlockSpec((F, F), full),
            pl.BlockSpec((1, F), full),
            pl.BlockSpec((BR, F), row),
            pl.BlockSpec((NC, BR, 1), lambda i: (0, i, 0)),
        ],
        out_specs=[
            pl.BlockSpec((BR, F), row),
            pl.BlockSpec((BR, F), row),
            pl.BlockSpec((BR, F), row),
            pl.BlockSpec((BR, 1), row),
            pl.BlockSpec((F, F), full),
        ],
        out_shape=[
            jax.ShapeDtypeStruct((N, F), jnp.float32),
            jax.ShapeDtypeStruct((N, F), jnp.float32),
            jax.ShapeDtypeStruct((N, F), jnp.float32),
            jax.ShapeDtypeStruct((N, 1), jnp.float32),
            jax.ShapeDtypeStruct((F, F), jnp.float32),
        ],
    )(x0, x1, wm, bm, w1, wp, bp, wg, dparts)


def _tc_x_kernel(g_ref, hg_ref, bg_ref, xg_ref):
    a = g_ref[0] + g_ref[1]
    degg = jnp.sum(a, axis=1, keepdims=True) + 1.0
    dinvg = lax.rsqrt(degg)
    rows = lax.broadcasted_iota(jnp.int32, (F, F), 0)
    cols = lax.broadcasted_iota(jnp.int32, (F, F), 1)
    eye = jnp.where(rows == cols, 1.0, 0.0)
    ahat = dinvg * (a + eye) * dinvg.reshape(1, F)
    xg_ref[...] = jax.nn.relu(_dot(ahat, hg_ref[...]) + bg_ref[...])


def _tc_x(gcnt, hg, bg):
    return pl.pallas_call(
        _tc_x_kernel,
        out_shape=jax.ShapeDtypeStruct((F, F), jnp.float32),
    )(gcnt, hg, bg)


def _tc_d_kernel(a_ref, hs1_ref, dinv_ref, b1_ref, w2_ref, hs2_ref):
    dinv = dinv_ref[...]
    out1 = dinv * (a_ref[0] + a_ref[1] + hs1_ref[...]) + b1_ref[...]
    t2 = _lrelu(out1)
    hs2_ref[...] = dinv * _dot(t2, w2_ref[...])


def _tc_d(agg, hs1, dinv, b1, w2):
    row = lambda i: (i, 0)
    full = lambda i: (0, 0)
    return pl.pallas_call(
        _tc_d_kernel,
        grid=(GRID,),
        in_specs=[
            pl.BlockSpec((NC, BR, F), lambda i: (0, i, 0)),
            pl.BlockSpec((BR, F), row),
            pl.BlockSpec((BR, 1), row),
            pl.BlockSpec((1, F), full),
            pl.BlockSpec((F, F), full),
        ],
        out_specs=pl.BlockSpec((BR, F), row),
        out_shape=jax.ShapeDtypeStruct((N, F), jnp.float32),
    )(agg, hs1, dinv, b1, w2)


def _tc_f_kernel(a_ref, hs2_ref, dinv_ref, b2_ref, w3_ref, b3_ref, wp_ref,
                 bp_ref, p1_ref, xg_ref, emb2_ref, z1_ref, z2_ref):
    dinv = dinv_ref[...]
    out2 = dinv * (a_ref[0] + a_ref[1] + hs2_ref[...]) + b2_ref[...]
    h2a = _lrelu(out2)
    h2 = _lrelu(_dot(h2a, w3_ref[...]) + b3_ref[...])
    emb2_ref[...] = _l2n(h2)
    p2 = _dot(h2, wp_ref[...]) + bp_ref[...]
    xg = xg_ref[...]
    z1_ref[...] = _l2n(_dot(p1_ref[...], xg))
    z2_ref[...] = _l2n(_dot(p2, xg))


def _tc_f(agg, hs2, dinv, b2, w3, b3, wp, bp, p1, xg):
    row = lambda i: (i, 0)
    full = lambda i: (0, 0)
    sd = jax.ShapeDtypeStruct((N, F), jnp.float32)
    return pl.pallas_call(
        _tc_f_kernel,
        grid=(GRID,),
        in_specs=[
            pl.BlockSpec((NC, BR, F), lambda i: (0, i, 0)),
            pl.BlockSpec((BR, F), row),
            pl.BlockSpec((BR, 1), row),
            pl.BlockSpec((1, F), full),
            pl.BlockSpec((F, F), full),
            pl.BlockSpec((1, F), full),
            pl.BlockSpec((F, F), full),
            pl.BlockSpec((1, F), full),
            pl.BlockSpec((BR, F), row),
            pl.BlockSpec((F, F), full),
        ],
        out_specs=[pl.BlockSpec((BR, F), row)] * 3,
        out_shape=[sd, sd, sd],
    )(agg, hs2, dinv, b2, w3, b3, wp, bp, p1, xg)


def kernel(x0, x1, W_mlp, b_mlp, W1, b1, W2, b2, W3, b3, Wp, bp, Wg, bg,
           edge_index, edge_index_g):
    src = edge_index[0]
    dst = edge_index[1]
    gsrc = edge_index_g[0]
    gdst = edge_index_g[1]

    degp, gcntp = _sc_hist(dst, gsrc, gdst)
    dparts = degp.reshape(NC, NPAD, 1)

    emb1, p1, hs1, dinv, hg = _tc_b(
        x0, x1, W_mlp, b_mlp.reshape(1, F), W1, Wp, bp.reshape(1, F), Wg,
        dparts)

    xg = _tc_x(gcntp.reshape(NC, F, F), hg, bg.reshape(1, F))

    agg1 = _sc_agg(hs1, src, dst)
    hs2 = _tc_d(agg1, hs1, dinv, b1.reshape(1, F), W2)

    agg2 = _sc_agg(hs2, src, dst)
    emb2, z1, z2 = _tc_f(agg2, hs2, dinv, b2.reshape(1, F), W3,
                         b3.reshape(1, F), Wp, bp.reshape(1, F), p1, xg)

    return (emb1, emb2, z1, z2)


# speedup vs baseline: 30.5985x; 2.3592x over previous
import functools

import jax
import jax.numpy as jnp
from jax import lax
from jax.experimental import pallas as pl
from jax.experimental.pallas import tpu as pltpu
from jax.experimental.pallas import tpu_sc as plsc

N = 10000
E = 320000
F = 128
EG = 2048

NC = 2
NS = 16
NW = NC * NS

EPT = E // NW
K = 40
NCH = EPT // K

EGPT = EG // NW
GBINS = F * F
NPAD = 10240
HB = 10
HLA = 4

@functools.cache
def _mesh():
    return plsc.VectorSubcoreMesh(core_axis_name="c", subcore_axis_name="s",
                                  num_cores=NC, num_subcores=NS)


def _zero_fill(ref, nrows, ncols):
    zer = jnp.zeros((16,), jnp.float32)

    def body(i, _):
        for j in range(ncols // 16):
            ref[i, pl.ds(j * 16, 16)] = zer
        return 0

    lax.fori_loop(0, nrows, body, 0)


def _zero_fill_1d(ref, n):
    zer = jnp.zeros((16,), jnp.float32)

    def body(i, _):
        ref[pl.ds(i * 16, 16)] = zer
        return 0

    lax.fori_loop(0, n // 16, body, 0)


def _ones_fill_1d(ref, n):
    one = jnp.full((16,), 1.0, jnp.float32)
    for j in range(n // 16):
        ref[pl.ds(j * 16, 16)] = one


@functools.cache
def _get_sc_hist():
  return pl.kernel(
    _sc_hist_body,
    out_type=[
        jax.ShapeDtypeStruct((NC, NPAD), jnp.float32),
        jax.ShapeDtypeStruct((NC, GBINS), jnp.float32),
    ],
    mesh=_mesh(),
    scratch_types=[
        pltpu.VMEM_SHARED((NPAD,), jnp.float32),
        pltpu.VMEM_SHARED((GBINS,), jnp.float32),
        pltpu.VMEM((1024,), jnp.float32),
    ] + [pltpu.VMEM((K,), jnp.int32)] * HB + [
        pltpu.VMEM((48,), jnp.float32),
        pltpu.VMEM((EGPT,), jnp.int32),
        pltpu.VMEM((EGPT,), jnp.int32),
        pltpu.VMEM((EGPT,), jnp.int32),
        pltpu.VMEM((EGPT,), jnp.float32),
        pltpu.VMEM((1024,), jnp.float32),
        pltpu.SemaphoreType.DMA((HB,)),
        pltpu.SemaphoreType.DMA((HB,)),
    ],
  )


def _sc_hist_body(dst_hbm, gsrc_hbm, gdst_hbm, deg_out, gcnt_out,
             sh_deg, sh_g, zbuf, *rest):
    dbufs = list(rest[:HB])
    ones_v, gs_v, gd_v, gf_v, gones_v, obuf, isem, ssem = rest[HB:]
    cid = lax.axis_index("c")
    sid = lax.axis_index("s")
    wid = cid * NS + sid
    base = wid * EPT

    _zero_fill_1d(zbuf, 1024)
    _ones_fill_1d(ones_v, 48)
    _ones_fill_1d(gones_v, EGPT)

    def idx_start(c, s):
        off = pl.multiple_of(base + c * K, 8)
        pltpu.async_copy(dst_hbm.at[pl.ds(off, K)], dbufs[s], isem.at[s])

    def idx_wait(s):
        pltpu.make_async_copy(dst_hbm.at[pl.ds(0, K)], dbufs[s],
                              isem.at[s]).wait()

    def sc_start(s):
        pltpu.async_copy(ones_v.at[pl.ds(0, K)], sh_deg.at[dbufs[s]],
                         ssem.at[s], add=True)

    def sc_wait(s):
        pltpu.make_async_copy(dst_hbm.at[pl.ds(0, K)], dbufs[s],
                              ssem.at[s]).wait()

    for c0 in range(HLA):
        idx_start(c0, c0)

    @pl.when(sid < 10)
    def _():
        pltpu.sync_copy(zbuf, sh_deg.at[pl.ds(sid * 1024, 1024)])

    pltpu.sync_copy(zbuf, sh_g.at[pl.ds(sid * 1024, 1024)])
    plsc.subcore_barrier()

    def group(g, _):
        for b in range(HB):
            c = g * HB + b
            s_new = (b + HLA) % HB

            @pl.when(c >= HB - HLA)
            def _():
                sc_wait(s_new)

            @pl.when(c + HLA < NCH)
            def _():
                idx_start(c + HLA, s_new)

            idx_wait(b)
            sc_start(b)
        return 0

    lax.fori_loop(0, NCH // HB, group, 0)
    for e in range(HB - HLA):
        sc_wait((NCH - (HB - HLA) + e) % HB)

    goff = pl.multiple_of(wid * EGPT, 8)
    pltpu.sync_copy(gsrc_hbm.at[pl.ds(goff, EGPT)], gs_v)
    pltpu.sync_copy(gdst_hbm.at[pl.ds(goff, EGPT)], gd_v)
    for j in range(EGPT // 16):
        s = gs_v[pl.ds(j * 16, 16)]
        d = gd_v[pl.ds(j * 16, 16)]
        gf_v[pl.ds(j * 16, 16)] = d * F + s
    pltpu.sync_copy(gones_v, sh_g.at[gf_v], add=True)

    plsc.subcore_barrier()

    @pl.when(sid < 10)
    def _():
        pltpu.sync_copy(sh_deg.at[pl.ds(sid * 1024, 1024)], obuf)
        pltpu.sync_copy(obuf, deg_out.at[cid, pl.ds(sid * 1024, 1024)])

    pltpu.sync_copy(sh_g.at[pl.ds(sid * 1024, 1024)], obuf)
    pltpu.sync_copy(obuf, gcnt_out.at[cid, pl.ds(sid * 1024, 1024)])


AB = 5
ALA = 2

@functools.cache
def _get_sc_agg():
  return pl.kernel(
    _sc_agg_body,
    out_type=jax.ShapeDtypeStruct((NC, N, F), jnp.float32),
    mesh=_mesh(),
    scratch_types=[
        pltpu.VMEM_SHARED((N, F), jnp.float32),
        pltpu.VMEM((EPT,), jnp.int32),
        pltpu.VMEM((AB, K), jnp.int32),
        pltpu.VMEM((AB, K, F), jnp.float32),
        pltpu.SemaphoreType.DMA((AB,)),
        pltpu.SemaphoreType.DMA((AB,)),
        pltpu.SemaphoreType.DMA((AB,)),
    ],
  )


def _sc_agg_body(h_hbm, src_hbm, dst_hbm, agg_out, acc, src_v, dring,
            rows_v, isem, gsem, ssem):
    cid = lax.axis_index("c")
    sid = lax.axis_index("s")
    wid = cid * NS + sid
    base = wid * EPT
    row0 = sid * 1000

    def idx_start(c, s):
        off = pl.multiple_of(base + c * K, 8)
        pltpu.async_copy(dst_hbm.at[pl.ds(off, K)], dring.at[s], isem.at[s])

    def idx_wait(s):
        pltpu.make_async_copy(dst_hbm.at[pl.ds(0, K)], dring.at[s],
                              isem.at[s]).wait()

    def gat_start(c, s):
        voff = pl.multiple_of(c * K, 8)
        pltpu.async_copy(h_hbm.at[src_v.at[pl.ds(voff, K)]], rows_v.at[s],
                         gsem.at[s])

    def gat_wait(s):
        pltpu.make_async_copy(h_hbm.at[pl.ds(0, K)], rows_v.at[s],
                              gsem.at[s]).wait()

    def sc_start(s):
        pltpu.async_copy(rows_v.at[s], acc.at[dring.at[s]], ssem.at[s],
                         add=True)

    def sc_wait(s):
        pltpu.make_async_copy(h_hbm.at[pl.ds(0, K)], rows_v.at[s],
                              ssem.at[s]).wait()

    pltpu.sync_copy(src_hbm.at[pl.ds(base, EPT)], src_v)
    for c0 in range(ALA):
        idx_start(c0, c0)
        gat_start(c0, c0)

    _zero_fill(rows_v.at[AB - 1], K, F)

    @pl.when(sid < 10)
    def _():
        def zrow(t, _):
            pltpu.sync_copy(rows_v.at[AB - 1],
                            acc.at[pl.ds(row0 + t * K, K)])
            return 0
        lax.fori_loop(0, 1000 // K, zrow, 0)

    plsc.subcore_barrier()

    def group(g, _):
        for b in range(AB):
            c = g * AB + b
            s_new = (b + ALA) % AB

            @pl.when(c >= AB - ALA)
            def _():
                sc_wait(s_new)

            @pl.when(c + ALA < NCH)
            def _():
                idx_start(c + ALA, s_new)
                gat_start(c + ALA, s_new)

            idx_wait(b)
            gat_wait(b)
            sc_start(b)
        return 0

    lax.fori_loop(0, NCH // AB, group, 0)
    for e in range(AB - ALA):
        sc_wait((NCH - (AB - ALA) + e) % AB)
    plsc.subcore_barrier()

    @pl.when(sid < 10)
    def _():
        for t in range(1000 // K):
            s = t % 2
            r = row0 + t * K
            if t >= 2:
                pltpu.make_async_copy(h_hbm.at[pl.ds(0, K)],
                                      rows_v.at[s], gsem.at[s]).wait()
            pltpu.sync_copy(acc.at[pl.ds(r, K)], rows_v.at[s])
            pltpu.async_copy(rows_v.at[s], agg_out.at[cid, pl.ds(r, K)],
                             gsem.at[s])
        for s in range(2):
            pltpu.make_async_copy(h_hbm.at[pl.ds(0, K)], rows_v.at[s],
                                  gsem.at[s]).wait()


def _sc_hist(dst, gsrc, gdst):
    return _get_sc_hist()(dst, gsrc, gdst)


def _sc_agg(h, src, dst):
    return _get_sc_agg()(h, src, dst)


BR = 1000
GRID = N // BR

def _dot(a, b):
    return jnp.dot(a, b, preferred_element_type=jnp.float32)


def _l2n(x):
    n = jnp.sqrt(jnp.sum(x * x, axis=1, keepdims=True))
    return x / jnp.maximum(n, 1e-12)


def _lrelu(v):
    return jnp.where(v >= 0, v, 0.01 * v)


def _tc_b_kernel(x0_ref, x1_ref, wm_ref, bm_ref, w1_ref, wp_ref, bp_ref,
                 wg_ref, dp_ref, emb1_ref, p1_ref, hs1_ref, dinv_ref,
                 hg_ref):
    i = pl.program_id(0)
    deg = dp_ref[0] + dp_ref[1] + 1.0
    dinv = lax.rsqrt(deg)
    dinv_ref[...] = dinv
    h1 = jax.nn.relu(_dot(x0_ref[...], wm_ref[...]) + bm_ref[...])
    emb1_ref[...] = _l2n(h1)
    p1_ref[...] = _dot(h1, wp_ref[...]) + bp_ref[...]
    hs1_ref[...] = dinv * _dot(x1_ref[...], w1_ref[...])

    @pl.when(i == 0)
    def _():
        hg_ref[...] = jnp.zeros((F, F), jnp.float32)

    hg_ref[...] += lax.dot_general(
        x0_ref[...], wg_ref[...], (((0,), (0,)), ((), ())),
        preferred_element_type=jnp.float32)


def _tc_b(x0, x1, wm, bm, w1, wp, bp, wg, dparts):
    row = lambda i: (i, 0)
    full = lambda i: (0, 0)
    return pl.pallas_call(
        _tc_b_kernel,
        grid=(GRID,),
        in_specs=[
            pl.BlockSpec((BR, F), row),
            pl.BlockSpec((BR, F), row),
            pl.BlockSpec((F, F), full),
            pl.BlockSpec((1, F), full),
            pl.BlockSpec((F, F), full),
            pl.BlockSpec((F, F), full),
            pl.BlockSpec((1, F), full),
            pl.BlockSpec((BR, F), row),
            pl.BlockSpec((NC, BR, 1), lambda i: (0, i, 0)),
        ],
        out_specs=[
            pl.BlockSpec((BR, F), row),
            pl.BlockSpec((BR, F), row),
            pl.BlockSpec((BR, F), row),
            pl.BlockSpec((BR, 1), row),
            pl.BlockSpec((F, F), full),
        ],
        out_shape=[
            jax.ShapeDtypeStruct((N, F), jnp.float32),
            jax.ShapeDtypeStruct((N, F), jnp.float32),
            jax.ShapeDtypeStruct((N, F), jnp.float32),
            jax.ShapeDtypeStruct((N, 1), jnp.float32),
            jax.ShapeDtypeStruct((F, F), jnp.float32),
        ],
    )(x0, x1, wm, bm, w1, wp, bp, wg, dparts)


def _tc_x_kernel(g_ref, hg_ref, bg_ref, xg_ref):
    a = g_ref[0] + g_ref[1]
    degg = jnp.sum(a, axis=1, keepdims=True) + 1.0
    dinvg = lax.rsqrt(degg)
    rows = lax.broadcasted_iota(jnp.int32, (F, F), 0)
    cols = lax.broadcasted_iota(jnp.int32, (F, F), 1)
    eye = jnp.where(rows == cols, 1.0, 0.0)
    ahat = dinvg * (a + eye) * dinvg.reshape(1, F)
    xg_ref[...] = jax.nn.relu(_dot(ahat, hg_ref[...]) + bg_ref[...])


def _tc_x(gcnt, hg, bg):
    return pl.pallas_call(
        _tc_x_kernel,
        out_shape=jax.ShapeDtypeStruct((F, F), jnp.float32),
    )(gcnt, hg, bg)


def _tc_d_kernel(a_ref, hs1_ref, dinv_ref, b1_ref, w2_ref, hs2_ref):
    dinv = dinv_ref[...]
    out1 = dinv * (a_ref[0] + a_ref[1] + hs1_ref[...]) + b1_ref[...]
    t2 = _lrelu(out1)
    hs2_ref[...] = dinv * _dot(t2, w2_ref[...])


def _tc_d(agg, hs1, dinv, b1, w2):
    row = lambda i: (i, 0)
    full = lambda i: (0, 0)
    return pl.pallas_call(
        _tc_d_kernel,
        grid=(GRID,),
        in_specs=[
            pl.BlockSpec((NC, BR, F), lambda i: (0, i, 0)),
            pl.BlockSpec((BR, F), row),
            pl.BlockSpec((BR, 1), row),
            pl.BlockSpec((1, F), full),
            pl.BlockSpec((F, F), full),
        ],
        out_specs=pl.BlockSpec((BR, F), row),
        out_shape=jax.ShapeDtypeStruct((N, F), jnp.float32),
    )(agg, hs1, dinv, b1, w2)


def _tc_f_kernel(a_ref, hs2_ref, dinv_ref, b2_ref, w3_ref, b3_ref, wp_ref,
                 bp_ref, p1_ref, xg_ref, emb2_ref, z1_ref, z2_ref):
    dinv = dinv_ref[...]
    out2 = dinv * (a_ref[0] + a_ref[1] + hs2_ref[...]) + b2_ref[...]
    h2a = _lrelu(out2)
    h2 = _lrelu(_dot(h2a, w3_ref[...]) + b3_ref[...])
    emb2_ref[...] = _l2n(h2)
    p2 = _dot(h2, wp_ref[...]) + bp_ref[...]
    xg = xg_ref[...]
    z1_ref[...] = _l2n(_dot(p1_ref[...], xg))
    z2_ref[...] = _l2n(_dot(p2, xg))


def _tc_f(agg, hs2, dinv, b2, w3, b3, wp, bp, p1, xg):
    row = lambda i: (i, 0)
    full = lambda i: (0, 0)
    sd = jax.ShapeDtypeStruct((N, F), jnp.float32)
    return pl.pallas_call(
        _tc_f_kernel,
        grid=(GRID,),
        in_specs=[
            pl.BlockSpec((NC, BR, F), lambda i: (0, i, 0)),
            pl.BlockSpec((BR, F), row),
            pl.BlockSpec((BR, 1), row),
            pl.BlockSpec((1, F), full),
            pl.BlockSpec((F, F), full),
            pl.BlockSpec((1, F), full),
            pl.BlockSpec((F, F), full),
            pl.BlockSpec((1, F), full),
            pl.BlockSpec((BR, F), row),
            pl.BlockSpec((F, F), full),
        ],
        out_specs=[pl.BlockSpec((BR, F), row)] * 3,
        out_shape=[sd, sd, sd],
    )(agg, hs2, dinv, b2, w3, b3, wp, bp, p1, xg)


def kernel(x0, x1, W_mlp, b_mlp, W1, b1, W2, b2, W3, b3, Wp, bp, Wg, bg,
           edge_index, edge_index_g):
    src = edge_index[0]
    dst = edge_index[1]
    gsrc = edge_index_g[0]
    gdst = edge_index_g[1]

    degp, gcntp = _sc_hist(dst, gsrc, gdst)
    dparts = degp.reshape(NC, NPAD, 1)

    emb1, p1, hs1, dinv, hg = _tc_b(
        x0, x1, W_mlp, b_mlp.reshape(1, F), W1, Wp, bp.reshape(1, F), Wg,
        dparts)

    xg = _tc_x(gcntp.reshape(NC, F, F), hg, bg.reshape(1, F))

    agg1 = _sc_agg(hs1, src, dst)
    hs2 = _tc_d(agg1, hs1, dinv, b1.reshape(1, F), W2)

    agg2 = _sc_agg(hs2, src, dst)
    emb2, z1, z2 = _tc_f(agg2, hs2, dinv, b2.reshape(1, F), W3,
                         b3.reshape(1, F), Wp, bp.reshape(1, F), p1, xg)

    return (emb1, emb2, z1, z2)


# speedup vs baseline: 33.3263x; 1.0891x over previous
import functools

import jax
import jax.numpy as jnp
from jax import lax
from jax.experimental import pallas as pl
from jax.experimental.pallas import tpu as pltpu
from jax.experimental.pallas import tpu_sc as plsc

N = 10000
E = 320000
F = 128
EG = 2048

NC = 2
NS = 16
NW = NC * NS

EPT = E // NW
K = 40
NCH = EPT // K

EGPT = EG // NW
GBINS = F * F
NPAD = 10240
HB = 10
HLA = 4

@functools.cache
def _mesh():
    return plsc.VectorSubcoreMesh(core_axis_name="c", subcore_axis_name="s",
                                  num_cores=NC, num_subcores=NS)


def _zero_fill(ref, nrows, ncols):
    zer = jnp.zeros((16,), jnp.float32)

    def body(i, _):
        for j in range(ncols // 16):
            ref[i, pl.ds(j * 16, 16)] = zer
        return 0

    lax.fori_loop(0, nrows, body, 0)


def _zero_fill_1d(ref, n):
    zer = jnp.zeros((16,), jnp.float32)

    def body(i, _):
        ref[pl.ds(i * 16, 16)] = zer
        return 0

    lax.fori_loop(0, n // 16, body, 0)


def _ones_fill_1d(ref, n):
    one = jnp.full((16,), 1.0, jnp.float32)
    for j in range(n // 16):
        ref[pl.ds(j * 16, 16)] = one


@functools.cache
def _get_sc_hist():
  return pl.kernel(
    _sc_hist_body,
    out_type=[
        jax.ShapeDtypeStruct((NC, NPAD), jnp.float32),
        jax.ShapeDtypeStruct((NC, GBINS), jnp.float32),
    ],
    mesh=_mesh(),
    scratch_types=[
        pltpu.VMEM_SHARED((NPAD,), jnp.float32),
        pltpu.VMEM_SHARED((GBINS,), jnp.float32),
        pltpu.VMEM((1024,), jnp.float32),
    ] + [pltpu.VMEM((K,), jnp.int32)] * HB + [
        pltpu.VMEM((48,), jnp.float32),
        pltpu.VMEM((EGPT,), jnp.int32),
        pltpu.VMEM((EGPT,), jnp.int32),
        pltpu.VMEM((EGPT,), jnp.int32),
        pltpu.VMEM((EGPT,), jnp.float32),
        pltpu.VMEM((1024,), jnp.float32),
        pltpu.SemaphoreType.DMA((HB,)),
        pltpu.SemaphoreType.DMA((HB,)),
    ],
  )


def _sc_hist_body(dst_hbm, gsrc_hbm, gdst_hbm, deg_out, gcnt_out,
             sh_deg, sh_g, zbuf, *rest):
    dbufs = list(rest[:HB])
    ones_v, gs_v, gd_v, gf_v, gones_v, obuf, isem, ssem = rest[HB:]
    cid = lax.axis_index("c")
    sid = lax.axis_index("s")
    wid = cid * NS + sid
    base = wid * EPT

    _zero_fill_1d(zbuf, 1024)
    _ones_fill_1d(ones_v, 48)
    _ones_fill_1d(gones_v, EGPT)

    def idx_start(c, s):
        off = pl.multiple_of(base + c * K, 8)
        pltpu.async_copy(dst_hbm.at[pl.ds(off, K)], dbufs[s], isem.at[s])

    def idx_wait(s):
        pltpu.make_async_copy(dst_hbm.at[pl.ds(0, K)], dbufs[s],
                              isem.at[s]).wait()

    def sc_start(s):
        pltpu.async_copy(ones_v.at[pl.ds(0, K)], sh_deg.at[dbufs[s]],
                         ssem.at[s], add=True)

    def sc_wait(s):
        pltpu.make_async_copy(dst_hbm.at[pl.ds(0, K)], dbufs[s],
                              ssem.at[s]).wait()

    for c0 in range(HLA):
        idx_start(c0, c0)

    @pl.when(sid < 10)
    def _():
        pltpu.sync_copy(zbuf, sh_deg.at[pl.ds(sid * 1024, 1024)])

    pltpu.sync_copy(zbuf, sh_g.at[pl.ds(sid * 1024, 1024)])
    plsc.subcore_barrier()

    def group(g, _):
        for b in range(HB):
            c = g * HB + b
            s_new = (b + HLA) % HB

            @pl.when(c >= HB - HLA)
            def _():
                sc_wait(s_new)

            @pl.when(c + HLA < NCH)
            def _():
                idx_start(c + HLA, s_new)

            idx_wait(b)
            sc_start(b)
        return 0

    lax.fori_loop(0, NCH // HB, group, 0)
    for e in range(HB - HLA):
        sc_wait((NCH - (HB - HLA) + e) % HB)

    goff = pl.multiple_of(wid * EGPT, 8)
    pltpu.sync_copy(gsrc_hbm.at[pl.ds(goff, EGPT)], gs_v)
    pltpu.sync_copy(gdst_hbm.at[pl.ds(goff, EGPT)], gd_v)
    for j in range(EGPT // 16):
        s = gs_v[pl.ds(j * 16, 16)]
        d = gd_v[pl.ds(j * 16, 16)]
        gf_v[pl.ds(j * 16, 16)] = d * F + s
    pltpu.sync_copy(gones_v, sh_g.at[gf_v], add=True)

    plsc.subcore_barrier()

    @pl.when(sid < 10)
    def _():
        pltpu.sync_copy(sh_deg.at[pl.ds(sid * 1024, 1024)], obuf)
        pltpu.sync_copy(obuf, deg_out.at[cid, pl.ds(sid * 1024, 1024)])

    pltpu.sync_copy(sh_g.at[pl.ds(sid * 1024, 1024)], obuf)
    pltpu.sync_copy(obuf, gcnt_out.at[cid, pl.ds(sid * 1024, 1024)])


AB = 5
ALA = 3

@functools.cache
def _get_sc_agg():
  return pl.kernel(
    _sc_agg_body,
    out_type=jax.ShapeDtypeStruct((NC, N, F), jnp.float32),
    mesh=_mesh(),
    scratch_types=[
        pltpu.VMEM_SHARED((N, F), jnp.float32),
        pltpu.VMEM((EPT,), jnp.int32),
        pltpu.VMEM((AB, K), jnp.int32),
        pltpu.VMEM((AB, K, F), jnp.float32),
        pltpu.SemaphoreType.DMA((AB,)),
        pltpu.SemaphoreType.DMA((AB,)),
        pltpu.SemaphoreType.DMA((AB,)),
    ],
  )


def _sc_agg_body(h_hbm, src_hbm, dst_hbm, agg_out, acc, src_v, dring,
            rows_v, isem, gsem, ssem):
    cid = lax.axis_index("c")
    sid = lax.axis_index("s")
    wid = cid * NS + sid
    base = wid * EPT
    row0 = sid * 1000

    def idx_start(c, s):
        off = pl.multiple_of(base + c * K, 8)
        pltpu.async_copy(dst_hbm.at[pl.ds(off, K)], dring.at[s], isem.at[s])

    def idx_wait(s):
        pltpu.make_async_copy(dst_hbm.at[pl.ds(0, K)], dring.at[s],
                              isem.at[s]).wait()

    def gat_start(c, s):
        voff = pl.multiple_of(c * K, 8)
        pltpu.async_copy(h_hbm.at[src_v.at[pl.ds(voff, K)]], rows_v.at[s],
                         gsem.at[s])

    def gat_wait(s):
        pltpu.make_async_copy(h_hbm.at[pl.ds(0, K)], rows_v.at[s],
                              gsem.at[s]).wait()

    def sc_start(s):
        pltpu.async_copy(rows_v.at[s], acc.at[dring.at[s]], ssem.at[s],
                         add=True)

    def sc_wait(s):
        pltpu.make_async_copy(h_hbm.at[pl.ds(0, K)], rows_v.at[s],
                              ssem.at[s]).wait()

    pltpu.sync_copy(src_hbm.at[pl.ds(base, EPT)], src_v)
    for c0 in range(ALA):
        idx_start(c0, c0)
        gat_start(c0, c0)

    _zero_fill(rows_v.at[AB - 1], K, F)

    @pl.when(sid < 10)
    def _():
        def zrow(t, _):
            pltpu.sync_copy(rows_v.at[AB - 1],
                            acc.at[pl.ds(row0 + t * K, K)])
            return 0
        lax.fori_loop(0, 1000 // K, zrow, 0)

    plsc.subcore_barrier()

    def group(g, _):
        for b in range(AB):
            c = g * AB + b
            s_new = (b + ALA) % AB

            @pl.when(c >= AB - ALA)
            def _():
                sc_wait(s_new)

            @pl.when(c + ALA < NCH)
            def _():
                idx_start(c + ALA, s_new)
                gat_start(c + ALA, s_new)

            idx_wait(b)
            gat_wait(b)
            sc_start(b)
        return 0

    lax.fori_loop(0, NCH // AB, group, 0)
    for e in range(AB - ALA):
        sc_wait((NCH - (AB - ALA) + e) % AB)
    plsc.subcore_barrier()

    @pl.when(sid < 10)
    def _():
        for t in range(1000 // K):
            s = t % 2
            r = row0 + t * K
            if t >= 2:
                pltpu.make_async_copy(h_hbm.at[pl.ds(0, K)],
                                      rows_v.at[s], gsem.at[s]).wait()
            pltpu.sync_copy(acc.at[pl.ds(r, K)], rows_v.at[s])
            pltpu.async_copy(rows_v.at[s], agg_out.at[cid, pl.ds(r, K)],
                             gsem.at[s])
        for s in range(2):
            pltpu.make_async_copy(h_hbm.at[pl.ds(0, K)], rows_v.at[s],
                                  gsem.at[s]).wait()


def _sc_hist(dst, gsrc, gdst):
    return _get_sc_hist()(dst, gsrc, gdst)


def _sc_agg(h, src, dst):
    return _get_sc_agg()(h, src, dst)


BR = 1000
GRID = N // BR

def _dot(a, b):
    return jnp.dot(a, b, preferred_element_type=jnp.float32)


def _l2n(x):
    n = jnp.sqrt(jnp.sum(x * x, axis=1, keepdims=True))
    return x / jnp.maximum(n, 1e-12)


def _lrelu(v):
    return jnp.where(v >= 0, v, 0.01 * v)


def _tc_b1_kernel(x0_ref, wm_ref, bm_ref, wp_ref, bp_ref, wg_ref,
                  emb1_ref, p1_ref, hg_ref):
    i = pl.program_id(0)
    h1 = jax.nn.relu(_dot(x0_ref[...], wm_ref[...]) + bm_ref[...])
    emb1_ref[...] = _l2n(h1)
    p1_ref[...] = _dot(h1, wp_ref[...]) + bp_ref[...]

    @pl.when(i == 0)
    def _():
        hg_ref[...] = jnp.zeros((F, F), jnp.float32)

    hg_ref[...] += lax.dot_general(
        x0_ref[...], wg_ref[...], (((0,), (0,)), ((), ())),
        preferred_element_type=jnp.float32)


def _tc_b1(x0, wm, bm, wp, bp, wg):
    row = lambda i: (i, 0)
    full = lambda i: (0, 0)
    return pl.pallas_call(
        _tc_b1_kernel,
        grid=(GRID,),
        in_specs=[
            pl.BlockSpec((BR, F), row),
            pl.BlockSpec((F, F), full),
            pl.BlockSpec((1, F), full),
            pl.BlockSpec((F, F), full),
            pl.BlockSpec((1, F), full),
            pl.BlockSpec((BR, F), row),
        ],
        out_specs=[
            pl.BlockSpec((BR, F), row),
            pl.BlockSpec((BR, F), row),
            pl.BlockSpec((F, F), full),
        ],
        out_shape=[
            jax.ShapeDtypeStruct((N, F), jnp.float32),
            jax.ShapeDtypeStruct((N, F), jnp.float32),
            jax.ShapeDtypeStruct((F, F), jnp.float32),
        ],
    )(x0, wm, bm, wp, bp, wg)


def _tc_b2_kernel(x1_ref, w1_ref, dp_ref, hs1_ref, dinv_ref):
    deg = dp_ref[0] + dp_ref[1] + 1.0
    dinv = lax.rsqrt(deg)
    dinv_ref[...] = dinv
    hs1_ref[...] = dinv * _dot(x1_ref[...], w1_ref[...])


def _tc_b2(x1, w1, dparts):
    row = lambda i: (i, 0)
    full = lambda i: (0, 0)
    return pl.pallas_call(
        _tc_b2_kernel,
        grid=(GRID,),
        in_specs=[
            pl.BlockSpec((BR, F), row),
            pl.BlockSpec((F, F), full),
            pl.BlockSpec((NC, BR, 1), lambda i: (0, i, 0)),
        ],
        out_specs=[
            pl.BlockSpec((BR, F), row),
            pl.BlockSpec((BR, 1), row),
        ],
        out_shape=[
            jax.ShapeDtypeStruct((N, F), jnp.float32),
            jax.ShapeDtypeStruct((N, 1), jnp.float32),
        ],
    )(x1, w1, dparts)


def _tc_z_kernel(p_ref, xg_ref, z_ref):
    z_ref[...] = _l2n(_dot(p_ref[...], xg_ref[...]))


def _tc_z(p, xg):
    row = lambda i: (i, 0)
    full = lambda i: (0, 0)
    return pl.pallas_call(
        _tc_z_kernel,
        grid=(GRID,),
        in_specs=[
            pl.BlockSpec((BR, F), row),
            pl.BlockSpec((F, F), full),
        ],
        out_specs=pl.BlockSpec((BR, F), row),
        out_shape=jax.ShapeDtypeStruct((N, F), jnp.float32),
    )(p, xg)


def _tc_x_kernel(g_ref, hg_ref, bg_ref, xg_ref):
    a = g_ref[0] + g_ref[1]
    degg = jnp.sum(a, axis=1, keepdims=True) + 1.0
    dinvg = lax.rsqrt(degg)
    rows = lax.broadcasted_iota(jnp.int32, (F, F), 0)
    cols = lax.broadcasted_iota(jnp.int32, (F, F), 1)
    eye = jnp.where(rows == cols, 1.0, 0.0)
    ahat = dinvg * (a + eye) * dinvg.reshape(1, F)
    xg_ref[...] = jax.nn.relu(_dot(ahat, hg_ref[...]) + bg_ref[...])


def _tc_x(gcnt, hg, bg):
    return pl.pallas_call(
        _tc_x_kernel,
        out_shape=jax.ShapeDtypeStruct((F, F), jnp.float32),
    )(gcnt, hg, bg)


def _tc_d_kernel(a_ref, hs1_ref, dinv_ref, b1_ref, w2_ref, hs2_ref):
    dinv = dinv_ref[...]
    out1 = dinv * (a_ref[0] + a_ref[1] + hs1_ref[...]) + b1_ref[...]
    t2 = _lrelu(out1)
    hs2_ref[...] = dinv * _dot(t2, w2_ref[...])


def _tc_d(agg, hs1, dinv, b1, w2):
    row = lambda i: (i, 0)
    full = lambda i: (0, 0)
    return pl.pallas_call(
        _tc_d_kernel,
        grid=(GRID,),
        in_specs=[
            pl.BlockSpec((NC, BR, F), lambda i: (0, i, 0)),
            pl.BlockSpec((BR, F), row),
            pl.BlockSpec((BR, 1), row),
            pl.BlockSpec((1, F), full),
            pl.BlockSpec((F, F), full),
        ],
        out_specs=pl.BlockSpec((BR, F), row),
        out_shape=jax.ShapeDtypeStruct((N, F), jnp.float32),
    )(agg, hs1, dinv, b1, w2)


def _tc_f_kernel(a_ref, hs2_ref, dinv_ref, b2_ref, w3_ref, b3_ref, wp_ref,
                 bp_ref, xg_ref, emb2_ref, z2_ref):
    dinv = dinv_ref[...]
    out2 = dinv * (a_ref[0] + a_ref[1] + hs2_ref[...]) + b2_ref[...]
    h2a = _lrelu(out2)
    h2 = _lrelu(_dot(h2a, w3_ref[...]) + b3_ref[...])
    emb2_ref[...] = _l2n(h2)
    p2 = _dot(h2, wp_ref[...]) + bp_ref[...]
    z2_ref[...] = _l2n(_dot(p2, xg_ref[...]))


def _tc_f(agg, hs2, dinv, b2, w3, b3, wp, bp, xg):
    row = lambda i: (i, 0)
    full = lambda i: (0, 0)
    sd = jax.ShapeDtypeStruct((N, F), jnp.float32)
    return pl.pallas_call(
        _tc_f_kernel,
        grid=(GRID,),
        in_specs=[
            pl.BlockSpec((NC, BR, F), lambda i: (0, i, 0)),
            pl.BlockSpec((BR, F), row),
            pl.BlockSpec((BR, 1), row),
            pl.BlockSpec((1, F), full),
            pl.BlockSpec((F, F), full),
            pl.BlockSpec((1, F), full),
            pl.BlockSpec((F, F), full),
            pl.BlockSpec((1, F), full),
            pl.BlockSpec((F, F), full),
        ],
        out_specs=[pl.BlockSpec((BR, F), row)] * 2,
        out_shape=[sd, sd],
    )(agg, hs2, dinv, b2, w3, b3, wp, bp, xg)


def kernel(x0, x1, W_mlp, b_mlp, W1, b1, W2, b2, W3, b3, Wp, bp, Wg, bg,
           edge_index, edge_index_g):
    src = edge_index[0]
    dst = edge_index[1]
    gsrc = edge_index_g[0]
    gdst = edge_index_g[1]

    degp, gcntp = _sc_hist(dst, gsrc, gdst)
    dparts = degp.reshape(NC, NPAD, 1)

    emb1, p1, hg = _tc_b1(x0, W_mlp, b_mlp.reshape(1, F), Wp,
                          bp.reshape(1, F), Wg)
    hs1, dinv = _tc_b2(x1, W1, dparts)

    agg1 = _sc_agg(hs1, src, dst)
    xg = _tc_x(gcntp.reshape(NC, F, F), hg, bg.reshape(1, F))
    z1 = _tc_z(p1, xg)

    hs2 = _tc_d(agg1, hs1, dinv, b1.reshape(1, F), W2)

    agg2 = _sc_agg(hs2, src, dst)
    emb2, z2 = _tc_f(agg2, hs2, dinv, b2.reshape(1, F), W3,
                     b3.reshape(1, F), Wp, bp.reshape(1, F), xg)

    return (emb1, emb2, z1, z2)


# speedup vs baseline: 34.0915x; 1.0230x over previous
import functools

import jax
import jax.numpy as jnp
from jax import lax
from jax.experimental import pallas as pl
from jax.experimental.pallas import tpu as pltpu
from jax.experimental.pallas import tpu_sc as plsc

N = 10000
E = 320000
F = 128
EG = 2048

NC = 2
NS = 16
NW = NC * NS

EPT = E // NW
K = 40
NCH = EPT // K

EGPT = EG // NW
GBINS = F * F
NPAD = 10240
KH = 80
NCHH = EPT // KH
HB = 5
HLA = 3

@functools.cache
def _mesh():
    return plsc.VectorSubcoreMesh(core_axis_name="c", subcore_axis_name="s",
                                  num_cores=NC, num_subcores=NS)


def _zero_fill(ref, nrows, ncols):
    zer = jnp.zeros((16,), jnp.float32)

    def body(i, _):
        for j in range(ncols // 16):
            ref[i, pl.ds(j * 16, 16)] = zer
        return 0

    lax.fori_loop(0, nrows, body, 0)


def _zero_fill_1d(ref, n):
    zer = jnp.zeros((16,), jnp.float32)

    def body(i, _):
        ref[pl.ds(i * 16, 16)] = zer
        return 0

    lax.fori_loop(0, n // 16, body, 0)


def _ones_fill_1d(ref, n):
    one = jnp.full((16,), 1.0, jnp.float32)
    for j in range(n // 16):
        ref[pl.ds(j * 16, 16)] = one


@functools.cache
def _get_sc_hist():
  return pl.kernel(
    _sc_hist_body,
    out_type=[
        jax.ShapeDtypeStruct((NC, NPAD), jnp.float32),
        jax.ShapeDtypeStruct((NC, GBINS), jnp.float32),
    ],
    mesh=_mesh(),
    scratch_types=[
        pltpu.VMEM_SHARED((NPAD,), jnp.float32),
        pltpu.VMEM_SHARED((GBINS,), jnp.float32),
        pltpu.VMEM((1024,), jnp.float32),
    ] + [pltpu.VMEM((KH,), jnp.int32)] * HB + [
        pltpu.VMEM((KH,), jnp.float32),
        pltpu.VMEM((EGPT,), jnp.int32),
        pltpu.VMEM((EGPT,), jnp.int32),
        pltpu.VMEM((EGPT,), jnp.int32),
        pltpu.VMEM((EGPT,), jnp.float32),
        pltpu.VMEM((1024,), jnp.float32),
        pltpu.SemaphoreType.DMA((HB,)),
        pltpu.SemaphoreType.DMA((HB,)),
    ],
  )


def _sc_hist_body(dst_hbm, gsrc_hbm, gdst_hbm, deg_out, gcnt_out,
             sh_deg, sh_g, zbuf, *rest):
    dbufs = list(rest[:HB])
    ones_v, gs_v, gd_v, gf_v, gones_v, obuf, isem, ssem = rest[HB:]
    cid = lax.axis_index("c")
    sid = lax.axis_index("s")
    wid = cid * NS + sid
    base = wid * EPT

    _zero_fill_1d(zbuf, 1024)
    _ones_fill_1d(ones_v, KH)
    _ones_fill_1d(gones_v, EGPT)

    def idx_start(c, s):
        off = pl.multiple_of(base + c * KH, 8)
        pltpu.async_copy(dst_hbm.at[pl.ds(off, KH)], dbufs[s], isem.at[s])

    def idx_wait(s):
        pltpu.make_async_copy(dst_hbm.at[pl.ds(0, KH)], dbufs[s],
                              isem.at[s]).wait()

    def sc_start(s):
        pltpu.async_copy(ones_v, sh_deg.at[dbufs[s]], ssem.at[s], add=True)

    def sc_wait(s):
        pltpu.make_async_copy(dst_hbm.at[pl.ds(0, KH)], dbufs[s],
                              ssem.at[s]).wait()

    for c0 in range(HLA):
        idx_start(c0, c0)

    @pl.when(sid < 10)
    def _():
        pltpu.sync_copy(zbuf, sh_deg.at[pl.ds(sid * 1024, 1024)])

    pltpu.sync_copy(zbuf, sh_g.at[pl.ds(sid * 1024, 1024)])
    plsc.subcore_barrier()

    def group(g, _):
        for b in range(HB):
            c = g * HB + b
            s_new = (b + HLA) % HB

            @pl.when(c >= HB - HLA)
            def _():
                sc_wait(s_new)

            @pl.when(c + HLA < NCHH)
            def _():
                idx_start(c + HLA, s_new)

            idx_wait(b)
            sc_start(b)
        return 0

    lax.fori_loop(0, NCHH // HB, group, 0)
    for e in range(HB - HLA):
        sc_wait((NCHH - (HB - HLA) + e) % HB)

    goff = pl.multiple_of(wid * EGPT, 8)
    pltpu.sync_copy(gsrc_hbm.at[pl.ds(goff, EGPT)], gs_v)
    pltpu.sync_copy(gdst_hbm.at[pl.ds(goff, EGPT)], gd_v)
    for j in range(EGPT // 16):
        s = gs_v[pl.ds(j * 16, 16)]
        d = gd_v[pl.ds(j * 16, 16)]
        gf_v[pl.ds(j * 16, 16)] = d * F + s
    pltpu.sync_copy(gones_v, sh_g.at[gf_v], add=True)

    plsc.subcore_barrier()

    @pl.when(sid < 10)
    def _():
        pltpu.sync_copy(sh_deg.at[pl.ds(sid * 1024, 1024)], obuf)
        pltpu.sync_copy(obuf, deg_out.at[cid, pl.ds(sid * 1024, 1024)])

    pltpu.sync_copy(sh_g.at[pl.ds(sid * 1024, 1024)], obuf)
    pltpu.sync_copy(obuf, gcnt_out.at[cid, pl.ds(sid * 1024, 1024)])


AB = 5
ALA = 3

@functools.cache
def _get_sc_agg():
  return pl.kernel(
    _sc_agg_body,
    out_type=jax.ShapeDtypeStruct((NC, N, F), jnp.float32),
    mesh=_mesh(),
    scratch_types=[
        pltpu.VMEM_SHARED((N, F), jnp.float32),
        pltpu.VMEM((EPT,), jnp.int32),
        pltpu.VMEM((AB, K), jnp.int32),
        pltpu.VMEM((AB, K, F), jnp.float32),
        pltpu.SemaphoreType.DMA((AB,)),
        pltpu.SemaphoreType.DMA((AB,)),
        pltpu.SemaphoreType.DMA((AB,)),
    ],
  )


def _sc_agg_body(h_hbm, src_hbm, dst_hbm, agg_out, acc, src_v, dring,
            rows_v, isem, gsem, ssem):
    cid = lax.axis_index("c")
    sid = lax.axis_index("s")
    wid = cid * NS + sid
    base = wid * EPT
    row0 = sid * 1000

    def idx_start(c, s):
        off = pl.multiple_of(base + c * K, 8)
        pltpu.async_copy(dst_hbm.at[pl.ds(off, K)], dring.at[s], isem.at[s])

    def idx_wait(s):
        pltpu.make_async_copy(dst_hbm.at[pl.ds(0, K)], dring.at[s],
                              isem.at[s]).wait()

    def gat_start(c, s):
        voff = pl.multiple_of(c * K, 8)
        pltpu.async_copy(h_hbm.at[src_v.at[pl.ds(voff, K)]], rows_v.at[s],
                         gsem.at[s])

    def gat_wait(s):
        pltpu.make_async_copy(h_hbm.at[pl.ds(0, K)], rows_v.at[s],
                              gsem.at[s]).wait()

    def sc_start(s):
        pltpu.async_copy(rows_v.at[s], acc.at[dring.at[s]], ssem.at[s],
                         add=True)

    def sc_wait(s):
        pltpu.make_async_copy(h_hbm.at[pl.ds(0, K)], rows_v.at[s],
                              ssem.at[s]).wait()

    pltpu.sync_copy(src_hbm.at[pl.ds(base, EPT)], src_v)
    for c0 in range(ALA):
        idx_start(c0, c0)
        gat_start(c0, c0)

    _zero_fill(rows_v.at[AB - 1], K, F)

    @pl.when(sid < 10)
    def _():
        def zrow(t, _):
            pltpu.sync_copy(rows_v.at[AB - 1],
                            acc.at[pl.ds(row0 + t * K, K)])
            return 0
        lax.fori_loop(0, 1000 // K, zrow, 0)

    plsc.subcore_barrier()

    def group(g, _):
        for b in range(AB):
            c = g * AB + b
            s_new = (b + ALA) % AB

            @pl.when(c >= AB - ALA)
            def _():
                sc_wait(s_new)

            @pl.when(c + ALA < NCH)
            def _():
                idx_start(c + ALA, s_new)
                gat_start(c + ALA, s_new)

            idx_wait(b)
            gat_wait(b)
            sc_start(b)
        return 0

    lax.fori_loop(0, NCH // AB, group, 0)
    for e in range(AB - ALA):
        sc_wait((NCH - (AB - ALA) + e) % AB)
    plsc.subcore_barrier()

    @pl.when(sid < 10)
    def _():
        for t in range(1000 // K):
            s = t % 2
            r = row0 + t * K
            if t >= 2:
                pltpu.make_async_copy(h_hbm.at[pl.ds(0, K)],
                                      rows_v.at[s], gsem.at[s]).wait()
            pltpu.sync_copy(acc.at[pl.ds(r, K)], rows_v.at[s])
            pltpu.async_copy(rows_v.at[s], agg_out.at[cid, pl.ds(r, K)],
                             gsem.at[s])
        for s in range(2):
            pltpu.make_async_copy(h_hbm.at[pl.ds(0, K)], rows_v.at[s],
                                  gsem.at[s]).wait()


def _sc_hist(dst, gsrc, gdst):
    return _get_sc_hist()(dst, gsrc, gdst)


def _sc_agg(h, src, dst):
    return _get_sc_agg()(h, src, dst)


BR = 1000
GRID = N // BR

def _dot(a, b):
    return jnp.dot(a, b, preferred_element_type=jnp.float32)


def _l2n(x):
    n = jnp.sqrt(jnp.sum(x * x, axis=1, keepdims=True))
    return x / jnp.maximum(n, 1e-12)


def _lrelu(v):
    return jnp.where(v >= 0, v, 0.01 * v)


def _tc_b1_kernel(x0_ref, wm_ref, bm_ref, wp_ref, bp_ref, wg_ref,
                  emb1_ref, p1_ref, hg_ref):
    i = pl.program_id(0)
    h1 = jax.nn.relu(_dot(x0_ref[...], wm_ref[...]) + bm_ref[...])
    emb1_ref[...] = _l2n(h1)
    p1_ref[...] = _dot(h1, wp_ref[...]) + bp_ref[...]

    @pl.when(i == 0)
    def _():
        hg_ref[...] = jnp.zeros((F, F), jnp.float32)

    hg_ref[...] += lax.dot_general(
        x0_ref[...], wg_ref[...], (((0,), (0,)), ((), ())),
        preferred_element_type=jnp.float32)


def _tc_b1(x0, wm, bm, wp, bp, wg):
    row = lambda i: (i, 0)
    full = lambda i: (0, 0)
    return pl.pallas_call(
        _tc_b1_kernel,
        grid=(GRID,),
        in_specs=[
            pl.BlockSpec((BR, F), row),
            pl.BlockSpec((F, F), full),
            pl.BlockSpec((1, F), full),
            pl.BlockSpec((F, F), full),
            pl.BlockSpec((1, F), full),
            pl.BlockSpec((BR, F), row),
        ],
        out_specs=[
            pl.BlockSpec((BR, F), row),
            pl.BlockSpec((BR, F), row),
            pl.BlockSpec((F, F), full),
        ],
        out_shape=[
            jax.ShapeDtypeStruct((N, F), jnp.float32),
            jax.ShapeDtypeStruct((N, F), jnp.float32),
            jax.ShapeDtypeStruct((F, F), jnp.float32),
        ],
    )(x0, wm, bm, wp, bp, wg)


def _tc_mm_kernel(x_ref, w_ref, o_ref):
    o_ref[...] = _dot(x_ref[...], w_ref[...])


def _tc_mm(x, w):
    row = lambda i: (i, 0)
    full = lambda i: (0, 0)
    return pl.pallas_call(
        _tc_mm_kernel,
        grid=(GRID,),
        in_specs=[
            pl.BlockSpec((BR, F), row),
            pl.BlockSpec((F, F), full),
        ],
        out_specs=pl.BlockSpec((BR, F), row),
        out_shape=jax.ShapeDtypeStruct((N, F), jnp.float32),
    )(x, w)


def _tc_b2_kernel(z_ref, dp_ref, hs1_ref, dinv_ref):
    deg = dp_ref[0] + dp_ref[1] + 1.0
    dinv = lax.rsqrt(deg)
    dinv_ref[...] = dinv
    hs1_ref[...] = dinv * z_ref[...]


def _tc_b2(zraw, dparts):
    row = lambda i: (i, 0)
    return pl.pallas_call(
        _tc_b2_kernel,
        grid=(GRID,),
        in_specs=[
            pl.BlockSpec((BR, F), row),
            pl.BlockSpec((NC, BR, 1), lambda i: (0, i, 0)),
        ],
        out_specs=[
            pl.BlockSpec((BR, F), row),
            pl.BlockSpec((BR, 1), row),
        ],
        out_shape=[
            jax.ShapeDtypeStruct((N, F), jnp.float32),
            jax.ShapeDtypeStruct((N, 1), jnp.float32),
        ],
    )(zraw, dparts)


def _tc_z_kernel(p_ref, xg_ref, z_ref):
    z_ref[...] = _l2n(_dot(p_ref[...], xg_ref[...]))


def _tc_z(p, xg):
    row = lambda i: (i, 0)
    full = lambda i: (0, 0)
    return pl.pallas_call(
        _tc_z_kernel,
        grid=(GRID,),
        in_specs=[
            pl.BlockSpec((BR, F), row),
            pl.BlockSpec((F, F), full),
        ],
        out_specs=pl.BlockSpec((BR, F), row),
        out_shape=jax.ShapeDtypeStruct((N, F), jnp.float32),
    )(p, xg)


def _tc_x_kernel(g_ref, hg_ref, bg_ref, xg_ref):
    a = g_ref[0] + g_ref[1]
    degg = jnp.sum(a, axis=1, keepdims=True) + 1.0
    dinvg = lax.rsqrt(degg)
    rows = lax.broadcasted_iota(jnp.int32, (F, F), 0)
    cols = lax.broadcasted_iota(jnp.int32, (F, F), 1)
    eye = jnp.where(rows == cols, 1.0, 0.0)
    ahat = dinvg * (a + eye) * dinvg.reshape(1, F)
    xg_ref[...] = jax.nn.relu(_dot(ahat, hg_ref[...]) + bg_ref[...])


def _tc_x(gcnt, hg, bg):
    return pl.pallas_call(
        _tc_x_kernel,
        out_shape=jax.ShapeDtypeStruct((F, F), jnp.float32),
    )(gcnt, hg, bg)


def _tc_d_kernel(a_ref, hs1_ref, dinv_ref, b1_ref, w2_ref, hs2_ref):
    dinv = dinv_ref[...]
    out1 = dinv * (a_ref[0] + a_ref[1] + hs1_ref[...]) + b1_ref[...]
    t2 = _lrelu(out1)
    hs2_ref[...] = dinv * _dot(t2, w2_ref[...])


def _tc_d(agg, hs1, dinv, b1, w2):
    row = lambda i: (i, 0)
    full = lambda i: (0, 0)
    return pl.pallas_call(
        _tc_d_kernel,
        grid=(GRID,),
        in_specs=[
            pl.BlockSpec((NC, BR, F), lambda i: (0, i, 0)),
            pl.BlockSpec((BR, F), row),
            pl.BlockSpec((BR, 1), row),
            pl.BlockSpec((1, F), full),
            pl.BlockSpec((F, F), full),
        ],
        out_specs=pl.BlockSpec((BR, F), row),
        out_shape=jax.ShapeDtypeStruct((N, F), jnp.float32),
    )(agg, hs1, dinv, b1, w2)


def _tc_f_kernel(a_ref, hs2_ref, dinv_ref, b2_ref, w3_ref, b3_ref, wp_ref,
                 bp_ref, xg_ref, emb2_ref, z2_ref):
    dinv = dinv_ref[...]
    out2 = dinv * (a_ref[0] + a_ref[1] + hs2_ref[...]) + b2_ref[...]
    h2a = _lrelu(out2)
    h2 = _lrelu(_dot(h2a, w3_ref[...]) + b3_ref[...])
    emb2_ref[...] = _l2n(h2)
    p2 = _dot(h2, wp_ref[...]) + bp_ref[...]
    z2_ref[...] = _l2n(_dot(p2, xg_ref[...]))


def _tc_f(agg, hs2, dinv, b2, w3, b3, wp, bp, xg):
    row = lambda i: (i, 0)
    full = lambda i: (0, 0)
    sd = jax.ShapeDtypeStruct((N, F), jnp.float32)
    return pl.pallas_call(
        _tc_f_kernel,
        grid=(GRID,),
        in_specs=[
            pl.BlockSpec((NC, BR, F), lambda i: (0, i, 0)),
            pl.BlockSpec((BR, F), row),
            pl.BlockSpec((BR, 1), row),
            pl.BlockSpec((1, F), full),
            pl.BlockSpec((F, F), full),
            pl.BlockSpec((1, F), full),
            pl.BlockSpec((F, F), full),
            pl.BlockSpec((1, F), full),
            pl.BlockSpec((F, F), full),
        ],
        out_specs=[pl.BlockSpec((BR, F), row)] * 2,
        out_shape=[sd, sd],
    )(agg, hs2, dinv, b2, w3, b3, wp, bp, xg)


def kernel(x0, x1, W_mlp, b_mlp, W1, b1, W2, b2, W3, b3, Wp, bp, Wg, bg,
           edge_index, edge_index_g):
    src = edge_index[0]
    dst = edge_index[1]
    gsrc = edge_index_g[0]
    gdst = edge_index_g[1]

    z1raw = _tc_mm(x1, W1)

    degp, gcntp = _sc_hist(dst, gsrc, gdst)
    dparts = degp.reshape(NC, NPAD, 1)

    emb1, p1, hg = _tc_b1(x0, W_mlp, b_mlp.reshape(1, F), Wp,
                          bp.reshape(1, F), Wg)
    hs1, dinv = _tc_b2(z1raw, dparts)

    agg1 = _sc_agg(hs1, src, dst)
    xg = _tc_x(gcntp.reshape(NC, F, F), hg, bg.reshape(1, F))
    z1 = _tc_z(p1, xg)

    hs2 = _tc_d(agg1, hs1, dinv, b1.reshape(1, F), W2)

    agg2 = _sc_agg(hs2, src, dst)
    emb2, z2 = _tc_f(agg2, hs2, dinv, b2.reshape(1, F), W3,
                     b3.reshape(1, F), Wp, bp.reshape(1, F), xg)

    return (emb1, emb2, z1, z2)


# speedup vs baseline: 34.4333x; 1.0100x over previous
import functools

import jax
import jax.numpy as jnp
from jax import lax
from jax.experimental import pallas as pl
from jax.experimental.pallas import tpu as pltpu
from jax.experimental.pallas import tpu_sc as plsc

N = 10000
E = 320000
F = 128
EG = 2048

NC = 2
NS = 16
NW = NC * NS

EPT = E // NW
K = 40
NCH = EPT // K

EGPT = EG // NW
GBINS = F * F
NPAD = 10240
KH = 80
NCHH = EPT // KH
HB = 5
HLA = 3

@functools.cache
def _mesh():
    return plsc.VectorSubcoreMesh(core_axis_name="c", subcore_axis_name="s",
                                  num_cores=NC, num_subcores=NS)


def _zero_fill(ref, nrows, ncols):
    zer = jnp.zeros((16,), jnp.float32)

    def body(i, _):
        for j in range(ncols // 16):
            ref[i, pl.ds(j * 16, 16)] = zer
        return 0

    lax.fori_loop(0, nrows, body, 0)


def _zero_fill_1d(ref, n):
    zer = jnp.zeros((16,), jnp.float32)

    def body(i, _):
        ref[pl.ds(i * 16, 16)] = zer
        return 0

    lax.fori_loop(0, n // 16, body, 0)


def _ones_fill_1d(ref, n):
    one = jnp.full((16,), 1.0, jnp.float32)
    for j in range(n // 16):
        ref[pl.ds(j * 16, 16)] = one


KE = 128
NCHT = E // KE
CPT = NCHT // NW
CREM = NCHT - CPT * NW
GCH = EG // KE


@functools.cache
def _get_sc_hist():
  return pl.kernel(
    _sc_hist_body,
    out_type=[
        jax.ShapeDtypeStruct((NC, NPAD), jnp.float32),
        jax.ShapeDtypeStruct((NC, GBINS), jnp.float32),
    ],
    mesh=_mesh(),
    scratch_types=[
        pltpu.VMEM_SHARED((NPAD,), jnp.float32),
        pltpu.VMEM_SHARED((GBINS,), jnp.float32),
        pltpu.VMEM((1024,), jnp.float32),
    ] + [pltpu.VMEM((2, KE), jnp.int32)] * 3 + [
        pltpu.VMEM((KE,), jnp.float32),
        pltpu.VMEM((2, KE), jnp.int32),
        pltpu.VMEM((KE,), jnp.int32),
        pltpu.VMEM((1024,), jnp.float32),
        pltpu.SemaphoreType.DMA((3,)),
        pltpu.SemaphoreType.DMA((3,)),
    ],
  )


def _sc_hist_body(ei_hbm, eig_hbm, deg_out, gcnt_out,
             sh_deg, sh_g, zbuf, e0, e1, e2, ones_v, gbuf, gf_v,
             obuf, isem, ssem):
    ebufs = [e0, e1, e2]
    cid = lax.axis_index("c")
    sid = lax.axis_index("s")
    wid = cid * NS + sid
    nw = CPT + (wid < CREM).astype(jnp.int32)
    start = wid * CPT + jnp.minimum(wid, CREM)

    _zero_fill_1d(zbuf, 1024)
    _ones_fill_1d(ones_v, KE)

    def ei_start(c, s):
        off = pl.multiple_of((start + c) * KE, KE)
        pltpu.async_copy(ei_hbm.at[pl.ds(0, 2), pl.ds(off, KE)], ebufs[s],
                         isem.at[s])

    def ei_wait(s):
        pltpu.make_async_copy(ei_hbm.at[pl.ds(0, 2), pl.ds(0, KE)],
                              ebufs[s], isem.at[s]).wait()

    def sc_start(s):
        pltpu.async_copy(ones_v, sh_deg.at[ebufs[s].at[1]], ssem.at[s],
                         add=True)

    def sc_wait(s):
        pltpu.make_async_copy(deg_out.at[cid, pl.ds(0, KE)], ones_v,
                              ssem.at[s]).wait()

    ei_start(0, 0)
    ei_start(1, 1)

    @pl.when(sid < 10)
    def _():
        pltpu.sync_copy(zbuf, sh_deg.at[pl.ds(sid * 1024, 1024)])

    pltpu.sync_copy(zbuf, sh_g.at[pl.ds(sid * 1024, 1024)])
    plsc.subcore_barrier()

    def group(g, _):
        for b in range(3):
            c = g * 3 + b
            s_new = (b + 2) % 3

            @pl.when(c >= 1)
            def _():
                sc_wait(s_new)

            @pl.when(c + 2 < nw)
            def _():
                ei_start(c + 2, s_new)

            ei_wait(b)
            sc_start(b)
        return 0

    lax.fori_loop(0, CPT // 3, group, 0)
    for b in range(CPT - (CPT // 3) * 3):
        pass
    sc_wait((CPT - 1) % 3)

    @pl.when(wid < CREM)
    def _():
        s = CPT % 3
        ei_wait(s)
        sc_start(s)
        sc_wait(s)

    @pl.when(wid < GCH)
    def _():
        goff = pl.multiple_of(wid * KE, KE)
        pltpu.sync_copy(eig_hbm.at[pl.ds(0, 2), pl.ds(goff, KE)], gbuf)
        for j in range(KE // 16):
            s = gbuf[0, pl.ds(j * 16, 16)]
            d = gbuf[1, pl.ds(j * 16, 16)]
            gf_v[pl.ds(j * 16, 16)] = d * F + s
        pltpu.sync_copy(ones_v, sh_g.at[gf_v], add=True)

    plsc.subcore_barrier()

    @pl.when(sid < 10)
    def _():
        pltpu.sync_copy(sh_deg.at[pl.ds(sid * 1024, 1024)], obuf)
        pltpu.sync_copy(obuf, deg_out.at[cid, pl.ds(sid * 1024, 1024)])

    pltpu.sync_copy(sh_g.at[pl.ds(sid * 1024, 1024)], obuf)
    pltpu.sync_copy(obuf, gcnt_out.at[cid, pl.ds(sid * 1024, 1024)])


@functools.cache
def _get_sc_agg():
  return pl.kernel(
    _sc_agg_body,
    out_type=jax.ShapeDtypeStruct((NC, N, F), jnp.float32),
    mesh=_mesh(),
    scratch_types=[
        pltpu.VMEM_SHARED((N, F), jnp.float32),
    ] + [pltpu.VMEM((2, KE), jnp.int32)] * 3 + [
        pltpu.VMEM((3, KE, F), jnp.float32),
        pltpu.SemaphoreType.DMA((3,)),
        pltpu.SemaphoreType.DMA((3,)),
        pltpu.SemaphoreType.DMA((3,)),
    ],
  )


def _sc_agg_body(h_hbm, ei_hbm, agg_out, acc, e0, e1, e2,
            rows_v, isem, gsem, ssem):
    ebufs = [e0, e1, e2]
    cid = lax.axis_index("c")
    sid = lax.axis_index("s")
    wid = cid * NS + sid
    nw = CPT + (wid < CREM).astype(jnp.int32)
    start = wid * CPT + jnp.minimum(wid, CREM)
    row0 = sid * 1000

    def ei_start(c, s):
        off = pl.multiple_of((start + c) * KE, KE)
        pltpu.async_copy(ei_hbm.at[pl.ds(0, 2), pl.ds(off, KE)], ebufs[s],
                         isem.at[s])

    def ei_wait(s):
        pltpu.make_async_copy(ei_hbm.at[pl.ds(0, 2), pl.ds(0, KE)],
                              ebufs[s], isem.at[s]).wait()

    def gat_start(s):
        pltpu.async_copy(h_hbm.at[ebufs[s].at[0]], rows_v.at[s],
                         gsem.at[s])

    def gat_wait(s):
        pltpu.make_async_copy(h_hbm.at[pl.ds(0, KE)], rows_v.at[s],
                              gsem.at[s]).wait()

    def sc_start(s):
        pltpu.async_copy(rows_v.at[s], acc.at[ebufs[s].at[1]], ssem.at[s],
                         add=True)

    def sc_wait(s):
        pltpu.make_async_copy(h_hbm.at[pl.ds(0, KE)], rows_v.at[s],
                              ssem.at[s]).wait()

    ei_start(0, 0)
    ei_start(1, 1)
    ei_wait(0)
    gat_start(0)

    _zero_fill(rows_v.at[2], KE, F)

    @pl.when(sid < 10)
    def _():
        for t in range(7):
            pltpu.sync_copy(rows_v.at[2], acc.at[pl.ds(row0 + t * KE, KE)])
        pltpu.sync_copy(rows_v.at[2, pl.ds(0, 104)],
                        acc.at[pl.ds(row0 + 7 * KE, 104)])

    plsc.subcore_barrier()

    def group(g, _):
        for b in range(3):
            c = g * 3 + b

            @pl.when(c >= 1)
            def _():
                sc_wait((b + 2) % 3)

            @pl.when(c + 2 < nw)
            def _():
                ei_start(c + 2, (b + 2) % 3)

            @pl.when(c + 1 < nw)
            def _():
                ei_wait((b + 1) % 3)
                gat_start((b + 1) % 3)

            gat_wait(b)
            sc_start(b)
        return 0

    lax.fori_loop(0, CPT // 3, group, 0)
    sc_wait((CPT - 1) % 3)

    @pl.when(wid < CREM)
    def _():
        s = CPT % 3
        gat_wait(s)
        sc_start(s)
        sc_wait(s)

    plsc.subcore_barrier()

    @pl.when(sid < 10)
    def _():
        for t in range(8):
            s = t % 2
            r = row0 + t * KE
            sz = KE if t < 7 else 104
            if t >= 2:
                pltpu.make_async_copy(h_hbm.at[pl.ds(0, KE)],
                                      rows_v.at[s], gsem.at[s]).wait()
            pltpu.sync_copy(acc.at[pl.ds(r, sz)],
                            rows_v.at[s, pl.ds(0, sz)])
            pltpu.async_copy(rows_v.at[s, pl.ds(0, sz)],
                             agg_out.at[cid, pl.ds(r, sz)], gsem.at[s])
        for s in range(2):
            sz = 104 if s == 1 else KE
            pltpu.make_async_copy(h_hbm.at[pl.ds(0, sz)],
                                  rows_v.at[s, pl.ds(0, sz)],
                                  gsem.at[s]).wait()


def _sc_hist(ei, eig):
    return _get_sc_hist()(ei, eig)


def _sc_agg(h, ei):
    return _get_sc_agg()(h, ei)


BR = 1000
GRID = N // BR

def _dot(a, b):
    return jnp.dot(a, b, preferred_element_type=jnp.float32)


def _l2n(x):
    n = jnp.sqrt(jnp.sum(x * x, axis=1, keepdims=True))
    return x / jnp.maximum(n, 1e-12)


def _lrelu(v):
    return jnp.where(v >= 0, v, 0.01 * v)


def _tc_b1_kernel(x0_ref, wm_ref, bm_ref, wp_ref, bp_ref, wg_ref,
                  emb1_ref, p1_ref, hg_ref):
    i = pl.program_id(0)
    h1 = jax.nn.relu(_dot(x0_ref[...], wm_ref[...]) + bm_ref[...])
    emb1_ref[...] = _l2n(h1)
    p1_ref[...] = _dot(h1, wp_ref[...]) + bp_ref[...]

    @pl.when(i == 0)
    def _():
        hg_ref[...] = jnp.zeros((F, F), jnp.float32)

    hg_ref[...] += lax.dot_general(
        x0_ref[...], wg_ref[...], (((0,), (0,)), ((), ())),
        preferred_element_type=jnp.float32)


def _tc_b1(x0, wm, bm, wp, bp, wg):
    row = lambda i: (i, 0)
    full = lambda i: (0, 0)
    return pl.pallas_call(
        _tc_b1_kernel,
        grid=(GRID,),
        in_specs=[
            pl.BlockSpec((BR, F), row),
            pl.BlockSpec((F, F), full),
            pl.BlockSpec((1, F), full),
            pl.BlockSpec((F, F), full),
            pl.BlockSpec((1, F), full),
            pl.BlockSpec((BR, F), row),
        ],
        out_specs=[
            pl.BlockSpec((BR, F), row),
            pl.BlockSpec((BR, F), row),
            pl.BlockSpec((F, F), full),
        ],
        out_shape=[
            jax.ShapeDtypeStruct((N, F), jnp.float32),
            jax.ShapeDtypeStruct((N, F), jnp.float32),
            jax.ShapeDtypeStruct((F, F), jnp.float32),
        ],
    )(x0, wm, bm, wp, bp, wg)


def _tc_mm_kernel(x_ref, w_ref, o_ref):
    o_ref[...] = _dot(x_ref[...], w_ref[...])


def _tc_mm(x, w):
    row = lambda i: (i, 0)
    full = lambda i: (0, 0)
    return pl.pallas_call(
        _tc_mm_kernel,
        grid=(GRID,),
        in_specs=[
            pl.BlockSpec((BR, F), row),
            pl.BlockSpec((F, F), full),
        ],
        out_specs=pl.BlockSpec((BR, F), row),
        out_shape=jax.ShapeDtypeStruct((N, F), jnp.float32),
    )(x, w)


def _tc_b2_kernel(z_ref, dp_ref, hs1_ref, dinv_ref):
    deg = dp_ref[0] + dp_ref[1] + 1.0
    dinv = lax.rsqrt(deg)
    dinv_ref[...] = dinv
    hs1_ref[...] = dinv * z_ref[...]


def _tc_b2(zraw, dparts):
    row = lambda i: (i, 0)
    return pl.pallas_call(
        _tc_b2_kernel,
        grid=(GRID,),
        in_specs=[
            pl.BlockSpec((BR, F), row),
            pl.BlockSpec((NC, BR, 1), lambda i: (0, i, 0)),
        ],
        out_specs=[
            pl.BlockSpec((BR, F), row),
            pl.BlockSpec((BR, 1), row),
        ],
        out_shape=[
            jax.ShapeDtypeStruct((N, F), jnp.float32),
            jax.ShapeDtypeStruct((N, 1), jnp.float32),
        ],
    )(zraw, dparts)


def _tc_z_kernel(p_ref, xg_ref, z_ref):
    z_ref[...] = _l2n(_dot(p_ref[...], xg_ref[...]))


def _tc_z(p, xg):
    row = lambda i: (i, 0)
    full = lambda i: (0, 0)
    return pl.pallas_call(
        _tc_z_kernel,
        grid=(GRID,),
        in_specs=[
            pl.BlockSpec((BR, F), row),
            pl.BlockSpec((F, F), full),
        ],
        out_specs=pl.BlockSpec((BR, F), row),
        out_shape=jax.ShapeDtypeStruct((N, F), jnp.float32),
    )(p, xg)


def _tc_x_kernel(g_ref, hg_ref, bg_ref, xg_ref):
    a = g_ref[0] + g_ref[1]
    degg = jnp.sum(a, axis=1, keepdims=True) + 1.0
    dinvg = lax.rsqrt(degg)
    rows = lax.broadcasted_iota(jnp.int32, (F, F), 0)
    cols = lax.broadcasted_iota(jnp.int32, (F, F), 1)
    eye = jnp.where(rows == cols, 1.0, 0.0)
    ahat = dinvg * (a + eye) * dinvg.reshape(1, F)
    xg_ref[...] = jax.nn.relu(_dot(ahat, hg_ref[...]) + bg_ref[...])


def _tc_x(gcnt, hg, bg):
    return pl.pallas_call(
        _tc_x_kernel,
        out_shape=jax.ShapeDtypeStruct((F, F), jnp.float32),
    )(gcnt, hg, bg)


def _tc_d_kernel(a_ref, hs1_ref, dinv_ref, b1_ref, w2_ref, hs2_ref):
    dinv = dinv_ref[...]
    out1 = dinv * (a_ref[0] + a_ref[1] + hs1_ref[...]) + b1_ref[...]
    t2 = _lrelu(out1)
    hs2_ref[...] = dinv * _dot(t2, w2_ref[...])


def _tc_d(agg, hs1, dinv, b1, w2):
    row = lambda i: (i, 0)
    full = lambda i: (0, 0)
    return pl.pallas_call(
        _tc_d_kernel,
        grid=(GRID,),
        in_specs=[
            pl.BlockSpec((NC, BR, F), lambda i: (0, i, 0)),
            pl.BlockSpec((BR, F), row),
            pl.BlockSpec((BR, 1), row),
            pl.BlockSpec((1, F), full),
            pl.BlockSpec((F, F), full),
        ],
        out_specs=pl.BlockSpec((BR, F), row),
        out_shape=jax.ShapeDtypeStruct((N, F), jnp.float32),
    )(agg, hs1, dinv, b1, w2)


def _tc_f_kernel(a_ref, hs2_ref, dinv_ref, b2_ref, w3_ref, b3_ref, wp_ref,
                 bp_ref, xg_ref, emb2_ref, z2_ref):
    dinv = dinv_ref[...]
    out2 = dinv * (a_ref[0] + a_ref[1] + hs2_ref[...]) + b2_ref[...]
    h2a = _lrelu(out2)
    h2 = _lrelu(_dot(h2a, w3_ref[...]) + b3_ref[...])
    emb2_ref[...] = _l2n(h2)
    p2 = _dot(h2, wp_ref[...]) + bp_ref[...]
    z2_ref[...] = _l2n(_dot(p2, xg_ref[...]))


def _tc_f(agg, hs2, dinv, b2, w3, b3, wp, bp, xg):
    row = lambda i: (i, 0)
    full = lambda i: (0, 0)
    sd = jax.ShapeDtypeStruct((N, F), jnp.float32)
    return pl.pallas_call(
        _tc_f_kernel,
        grid=(GRID,),
        in_specs=[
            pl.BlockSpec((NC, BR, F), lambda i: (0, i, 0)),
            pl.BlockSpec((BR, F), row),
            pl.BlockSpec((BR, 1), row),
            pl.BlockSpec((1, F), full),
            pl.BlockSpec((F, F), full),
            pl.BlockSpec((1, F), full),
            pl.BlockSpec((F, F), full),
            pl.BlockSpec((1, F), full),
            pl.BlockSpec((F, F), full),
        ],
        out_specs=[pl.BlockSpec((BR, F), row)] * 2,
        out_shape=[sd, sd],
    )(agg, hs2, dinv, b2, w3, b3, wp, bp, xg)


def kernel(x0, x1, W_mlp, b_mlp, W1, b1, W2, b2, W3, b3, Wp, bp, Wg, bg,
           edge_index, edge_index_g):
    z1raw = _tc_mm(x1, W1)

    degp, gcntp = _sc_hist(edge_index, edge_index_g)
    dparts = degp.reshape(NC, NPAD, 1)

    emb1, p1, hg = _tc_b1(x0, W_mlp, b_mlp.reshape(1, F), Wp,
                          bp.reshape(1, F), Wg)
    hs1, dinv = _tc_b2(z1raw, dparts)

    agg1 = _sc_agg(hs1, edge_index)
    xg = _tc_x(gcntp.reshape(NC, F, F), hg, bg.reshape(1, F))
    z1 = _tc_z(p1, xg)

    hs2 = _tc_d(agg1, hs1, dinv, b1.reshape(1, F), W2)

    agg2 = _sc_agg(hs2, edge_index)
    emb2, z2 = _tc_f(agg2, hs2, dinv, b2.reshape(1, F), W3,
                     b3.reshape(1, F), Wp, bp.reshape(1, F), xg)

    return (emb1, emb2, z1, z2)
